# Initial kernel scaffold; baseline (speedup 1.0000x reference)
import jax
import jax.numpy as jnp
from jax.experimental import pallas as pl


def kernel(user_text, user_feats, graph_node_features, graph_edge_index, tweet_emb, fc_w1, fc_b1, fc_w2, fc_b2, gru_wi0, gru_wh0, gru_bi0, gru_bh0, gru_wi1, gru_wh1, gru_bi1, gru_bh1, lin1_w, att_src1, att_dst1, bias1, lin2_w, att_src2, att_dst2, bias2):
    raise NotImplementedError("write your pallas kernel here")


# speedup vs baseline: 1.0052x; 1.0052x over previous
import functools
import jax
import jax.numpy as jnp
from jax.experimental import pallas as pl
from jax.experimental.pallas import tpu as pltpu

N_USERS = 2000
N = 10000
T = 20
D = 100
H = 100
BATCH = 1024
E = 160000

GRU_BN = 1000


def _gru_body(tw_ref, h0_ref, wi0_ref, wh0_ref, b0_ref, wi1_ref, wh1_ref,
              b1_ref, hn_ref):
    h1 = h0_ref[0]
    h2 = h0_ref[1]

    def step(t, carry):
        h1, h2 = carry

        def cell(x, h, wi_t, wh_t, bi, bh):
            gi = jnp.dot(x, wi_t, preferred_element_type=jnp.float32) + bi
            gh = jnp.dot(h, wh_t, preferred_element_type=jnp.float32) + bh
            i_r, i_z, i_n = gi[:, :H], gi[:, H:2 * H], gi[:, 2 * H:]
            h_r, h_z, h_n = gh[:, :H], gh[:, H:2 * H], gh[:, 2 * H:]
            r = jax.nn.sigmoid(i_r + h_r)
            z = jax.nn.sigmoid(i_z + h_z)
            n = jnp.tanh(i_n + r * h_n)
            return (1.0 - z) * n + z * h

        x = tw_ref[t]
        h1 = cell(x, h1, wi0_ref[...], wh0_ref[...], b0_ref[0], b0_ref[1])
        h2 = cell(h1, h2, wi1_ref[...], wh1_ref[...], b1_ref[0], b1_ref[1])
        return (h1, h2)

    h1, h2 = jax.lax.fori_loop(0, T, step, (h1, h2))
    hn_ref[...] = h2


def _gru_pallas(tw, h0, wi0_t, wh0_t, b0, wi1_t, wh1_t, b1):
    nt = tw.shape[1]
    grid = nt // GRU_BN
    return pl.pallas_call(
        _gru_body,
        grid=(grid,),
        in_specs=[
            pl.BlockSpec((T, GRU_BN, D), lambda i: (0, i, 0)),
            pl.BlockSpec((2, GRU_BN, H), lambda i: (0, i, 0)),
            pl.BlockSpec((D, 3 * H), lambda i: (0, 0)),
            pl.BlockSpec((H, 3 * H), lambda i: (0, 0)),
            pl.BlockSpec((2, 3 * H), lambda i: (0, 0)),
            pl.BlockSpec((H, 3 * H), lambda i: (0, 0)),
            pl.BlockSpec((H, 3 * H), lambda i: (0, 0)),
            pl.BlockSpec((2, 3 * H), lambda i: (0, 0)),
        ],
        out_specs=pl.BlockSpec((GRU_BN, H), lambda i: (i, 0)),
        out_shape=jax.ShapeDtypeStruct((nt, H), jnp.float32),
    )(tw, h0, wi0_t, wh0_t, b0, wi1_t, wh1_t, b1)


def _gat_conv(x, src, dst, lin_w, att_src, att_dst, bias, heads, out_ch,
              concat):
    n = x.shape[0]
    xh = (x @ lin_w).reshape(n, heads, out_ch)
    a_src = jnp.sum(xh * att_src, axis=-1)
    a_dst = jnp.sum(xh * att_dst, axis=-1)
    alpha = jax.nn.leaky_relu(a_src[src] + a_dst[dst], 0.2)
    amax = jax.ops.segment_max(alpha, dst, num_segments=n)
    amax = jnp.where(jnp.isfinite(amax), amax, 0.0)
    ex = jnp.exp(alpha - amax[dst])
    denom = jax.ops.segment_sum(ex, dst, num_segments=n)
    coef = ex / (denom[dst] + 1e-16)
    msg = xh[src] * coef[..., None]
    out = jax.ops.segment_sum(msg, dst, num_segments=n)
    out = out.reshape(n, heads * out_ch) if concat else out.mean(axis=1)
    return out + bias


def kernel(user_text, user_feats, graph_node_features, graph_edge_index,
           tweet_emb, fc_w1, fc_b1, fc_w2, fc_b2, gru_wi0, gru_wh0, gru_bi0,
           gru_bh0, gru_wi1, gru_wh1, gru_bi1, gru_bh1, lin1_w, att_src1,
           att_dst1, bias1, lin2_w, att_src2, att_dst2, bias2):
    user_embedding = jax.nn.relu(user_feats @ fc_w1 + fc_b1) @ fc_w2 + fc_b2

    tw = jnp.take(tweet_emb, graph_node_features, axis=0)
    tw = jnp.transpose(tw, (1, 0, 2))
    h0 = jax.random.normal(jax.random.key(42), (2, tw.shape[1], H),
                           dtype=jnp.float32)
    h0b = jnp.transpose(h0, (0, 1, 2))

    b0 = jnp.stack([gru_bi0, gru_bh0])
    b1 = jnp.stack([gru_bi1, gru_bh1])
    hn = _gru_pallas(tw, h0b, gru_wi0.T, gru_wh0.T, b0, gru_wi1.T, gru_wh1.T,
                     b1)

    x_input = jnp.concatenate([hn[:BATCH], user_embedding, hn[BATCH:]], axis=0)

    loop = jnp.arange(x_input.shape[0], dtype=graph_edge_index.dtype)
    src = jnp.concatenate([graph_edge_index[0], loop])
    dst = jnp.concatenate([graph_edge_index[1], loop])
    x = jax.nn.relu(
        _gat_conv(x_input, src, dst, lin1_w, att_src1, att_dst1, bias1, 8, 64,
                  True))
    x = jax.nn.relu(
        _gat_conv(x, src, dst, lin2_w, att_src2, att_dst2, bias2, 1, 100,
                  False))
    return x


# speedup vs baseline: 10.6947x; 10.6392x over previous
import functools
import jax
import jax.numpy as jnp
from jax import lax
from jax.experimental import pallas as pl
from jax.experimental.pallas import tpu as pltpu
from jax.experimental.pallas import tpu_sc as plsc

N = 10000
T = 20
D = 100
H = 100
BATCH = 1024
E = 160000

NC = 2
NS = 16
NW = NC * NS
KB = 128
E_PAD = 172032
EPW = E_PAD // NW
NBLK = EPW // KB
N_PAD = 10240
ROWS_PT = N_PAD // NS
ZR = 128

GRU_BN = 1000


def _gru_body(tw_ref, h0_ref, wi0_ref, wh0_ref, b0_ref, wi1_ref, wh1_ref,
              b1_ref, hn_ref):
    h1 = h0_ref[0]
    h2 = h0_ref[1]

    def step(t, carry):
        h1, h2 = carry

        def cell(x, h, wi_t, wh_t, bi, bh):
            gi = jnp.dot(x, wi_t, preferred_element_type=jnp.float32) + bi
            gh = jnp.dot(h, wh_t, preferred_element_type=jnp.float32) + bh
            r = jax.nn.sigmoid(gi[:, :H] + gh[:, :H])
            z = jax.nn.sigmoid(gi[:, H:2 * H] + gh[:, H:2 * H])
            n = jnp.tanh(gi[:, 2 * H:] + r * gh[:, 2 * H:])
            return (1.0 - z) * n + z * h

        x = tw_ref[t]
        h1 = cell(x, h1, wi0_ref[...], wh0_ref[...], b0_ref[0], b0_ref[1])
        h2 = cell(h1, h2, wi1_ref[...], wh1_ref[...], b1_ref[0], b1_ref[1])
        return (h1, h2)

    h1, h2 = jax.lax.fori_loop(0, T, step, (h1, h2))
    hn_ref[...] = h2


def _gru_pallas(tw, h0, wi0_t, wh0_t, b0, wi1_t, wh1_t, b1):
    nt = tw.shape[1]
    return pl.pallas_call(
        _gru_body,
        grid=(nt // GRU_BN,),
        in_specs=[
            pl.BlockSpec((T, GRU_BN, D), lambda i: (0, i, 0)),
            pl.BlockSpec((2, GRU_BN, H), lambda i: (0, i, 0)),
            pl.BlockSpec((D, 3 * H), lambda i: (0, 0)),
            pl.BlockSpec((H, 3 * H), lambda i: (0, 0)),
            pl.BlockSpec((2, 3 * H), lambda i: (0, 0)),
            pl.BlockSpec((H, 3 * H), lambda i: (0, 0)),
            pl.BlockSpec((H, 3 * H), lambda i: (0, 0)),
            pl.BlockSpec((2, 3 * H), lambda i: (0, 0)),
        ],
        out_specs=pl.BlockSpec((GRU_BN, H), lambda i: (i, 0)),
        out_shape=jax.ShapeDtypeStruct((nt, H), jnp.float32),
    )(tw, h0, wi0_t, wh0_t, b0, wi1_t, wh1_t, b1)


P_BN = 2000


def _p1_body(x_ref, w_ref, asr_ref, adr_ref, xh_ref, as_ref, ad_ref, ms_ref,
             md_ref):
    xh = jnp.dot(x_ref[...], w_ref[...], preferred_element_type=jnp.float32)
    a_s, a_d = [], []
    for h in range(8):
        xh_h = xh[:, h * 64:(h + 1) * 64]
        xh_ref[h] = xh_h
        a_s.append(jnp.sum(xh_h * asr_ref[h][None, :], axis=1)[:, None])
        a_d.append(jnp.sum(xh_h * adr_ref[h][None, :], axis=1)[:, None])
    a_s = jnp.concatenate(a_s, axis=1)
    a_d = jnp.concatenate(a_d, axis=1)
    as_ref[...] = a_s
    ad_ref[...] = a_d
    ms_ref[...] = jnp.full((1, 8, 128), jnp.max(a_s), dtype=jnp.float32)
    md_ref[...] = jnp.full((1, 8, 128), jnp.max(a_d), dtype=jnp.float32)


def _p1_pallas(x, lin_w, att_src, att_dst):
    n = x.shape[0]
    g = n // P_BN
    return pl.pallas_call(
        _p1_body,
        grid=(g,),
        in_specs=[
            pl.BlockSpec((P_BN, D), lambda i: (i, 0)),
            pl.BlockSpec((D, 512), lambda i: (0, 0)),
            pl.BlockSpec((8, 64), lambda i: (0, 0)),
            pl.BlockSpec((8, 64), lambda i: (0, 0)),
        ],
        out_specs=[
            pl.BlockSpec((8, P_BN, 64), lambda i: (0, i, 0)),
            pl.BlockSpec((P_BN, 8), lambda i: (i, 0)),
            pl.BlockSpec((P_BN, 8), lambda i: (i, 0)),
            pl.BlockSpec((1, 8, 128), lambda i: (i, 0, 0)),
            pl.BlockSpec((1, 8, 128), lambda i: (i, 0, 0)),
        ],
        out_shape=[
            jax.ShapeDtypeStruct((8, n, 64), jnp.float32),
            jax.ShapeDtypeStruct((n, 8), jnp.float32),
            jax.ShapeDtypeStruct((n, 8), jnp.float32),
            jax.ShapeDtypeStruct((g, 8, 128), jnp.float32),
            jax.ShapeDtypeStruct((g, 8, 128), jnp.float32),
        ],
    )(x, lin_w, att_src, att_dst)


def _make_edge_kernel(heads, c):
    mesh = plsc.VectorSubcoreMesh(core_axis_name="c", subcore_axis_name="s")

    @functools.partial(
        pl.kernel,
        mesh=mesh,
        compiler_params=pltpu.CompilerParams(use_tc_tiling_on_sc=False),
        out_type=[
            jax.ShapeDtypeStruct((NC, heads, N_PAD, c), jnp.float32),
            jax.ShapeDtypeStruct((NC, heads, N_PAD), jnp.float32),
        ],
        scratch_types=[
            pltpu.VMEM((NBLK, KB), jnp.int32),
            pltpu.VMEM((NBLK, KB), jnp.int32),
            pltpu.VMEM((KB,), jnp.int32),
            pltpu.VMEM((KB,), jnp.int32),
            pltpu.VMEM((KB,), jnp.float32),
            pltpu.VMEM((KB,), jnp.float32),
            pltpu.VMEM((KB,), jnp.float32),
            pltpu.VMEM((KB, c), jnp.float32),
            pltpu.VMEM((16,), jnp.float32),
            pltpu.VMEM((ZR, c), jnp.float32),
            pltpu.VMEM((ROWS_PT,), jnp.float32),
            pltpu.VMEM_SHARED((N_PAD, c), jnp.float32),
            pltpu.VMEM_SHARED((N_PAD,), jnp.float32),
            pltpu.SemaphoreType.DMA,
            pltpu.SemaphoreType.DMA,
            pltpu.SemaphoreType.DMA,
        ],
    )
    def edge_kernel(src_hbm, dst_hbm, xh_hbm, as_hbm, ad_hbm, m_hbm,
                    out_hbm, den_hbm,
                    srcv, dstv, idxs, idxd, asg, adg, exq, rowb, mv,
                    zrow, zden, out_sh, den_sh, sem0, sem1, sem2):
        cid = lax.axis_index("c")
        sid = lax.axis_index("s")
        wid = sid * NC + cid
        ebase = wid * EPW

        pltpu.sync_copy(src_hbm.at[wid], srcv)
        pltpu.sync_copy(dst_hbm.at[wid], dstv)
        pltpu.sync_copy(m_hbm, mv)

        zv = jnp.zeros((16,), jnp.float32)

        def zrow_body(r, _):
            for j in range(c // 16):
                zrow[r, pl.ds(j * 16, 16)] = zv
            return 0
        lax.fori_loop(0, ZR, zrow_body, 0)

        def zden_body(i, _):
            zden[pl.ds(i * 16, 16)] = zv
            return 0
        lax.fori_loop(0, ROWS_PT // 16, zden_body, 0)

        mvec = mv[...]
        lanes = lax.iota(jnp.int32, 16)

        def per_head(h, _):
            for z in range(ROWS_PT // ZR):
                pltpu.sync_copy(
                    zrow, out_sh.at[pl.ds(sid * ROWS_PT + z * ZR, ZR)])
            pltpu.sync_copy(zden, den_sh.at[pl.ds(sid * ROWS_PT, ROWS_PT)])
            plsc.subcore_barrier()

            def per_block(jb, _):
                def mk_idx(k, _):
                    sv = srcv[jb, pl.ds(k * 16, 16)]
                    dv = dstv[jb, pl.ds(k * 16, 16)]
                    idxs[pl.ds(k * 16, 16)] = sv + h * N
                    idxd[pl.ds(k * 16, 16)] = dv + h * N
                    return 0
                lax.fori_loop(0, KB // 16, mk_idx, 0)

                cp_s = pltpu.async_copy(as_hbm.at[idxs], asg, sem0)
                cp_d = pltpu.async_copy(ad_hbm.at[idxd], adg, sem1)
                cp_r = pltpu.async_copy(xh_hbm.at[idxs], rowb, sem2)
                cp_s.wait()
                cp_d.wait()

                def mk_ex(k, _):
                    al = asg[pl.ds(k * 16, 16)] + adg[pl.ds(k * 16, 16)]
                    al = jnp.where(al >= 0.0, al, al * 0.2)
                    ex = jnp.exp(al - mvec)
                    gid = ebase + jb * KB + k * 16 + lanes
                    ex = jnp.where(gid < E + N, ex, 0.0)
                    exq[pl.ds(k * 16, 16)] = ex
                    return 0
                lax.fori_loop(0, KB // 16, mk_ex, 0)

                pltpu.sync_copy(exq, den_sh.at[dstv.at[jb]], add=True)
                cp_r.wait()

                def scale(k, _):
                    exvec = exq[pl.ds(k * 16, 16)]
                    for e in range(16):
                        coef = jnp.full((16,), exvec[e], dtype=jnp.float32)
                        r = k * 16 + e
                        for j in range(c // 16):
                            rowb[r, pl.ds(j * 16, 16)] = (
                                rowb[r, pl.ds(j * 16, 16)] * coef)
                    return 0
                lax.fori_loop(0, KB // 16, scale, 0)

                pltpu.sync_copy(rowb, out_sh.at[dstv.at[jb]], add=True)
                return 0

            lax.fori_loop(0, NBLK, per_block, 0)
            plsc.subcore_barrier()

            pltpu.sync_copy(
                out_sh.at[pl.ds(sid * ROWS_PT, ROWS_PT)],
                out_hbm.at[cid, h, pl.ds(sid * ROWS_PT, ROWS_PT)])
            pltpu.sync_copy(
                den_sh.at[pl.ds(sid * ROWS_PT, ROWS_PT)],
                den_hbm.at[cid, h, pl.ds(sid * ROWS_PT, ROWS_PT)])
            return 0

        lax.fori_loop(0, heads, per_head, 0)

    return edge_kernel


_edge_l1 = _make_edge_kernel(8, 64)
_edge_l2 = _make_edge_kernel(1, 112)


def _p2_body(num_ref, den_ref, b1_ref, w2_ref, as2_ref, ad2_ref,
             xh2_ref, a2s_ref, a2d_ref, ms_ref, md_ref):
    x1 = []
    for h in range(8):
        dh = den_ref[:, h][:, None] + 1e-16
        x1.append(num_ref[h] / dh)
    x1 = jnp.concatenate(x1, axis=1)
    x1 = jax.nn.relu(x1 + b1_ref[...])
    xh2 = jnp.dot(x1, w2_ref[...], preferred_element_type=jnp.float32)
    a2s = jnp.sum(xh2 * as2_ref[...], axis=1)
    a2d = jnp.sum(xh2 * ad2_ref[...], axis=1)
    bn = xh2.shape[0]
    xh2_ref[...] = jnp.concatenate(
        [xh2, jnp.zeros((bn, 12), jnp.float32)], axis=1)
    a2s_ref[...] = a2s[:, None]
    a2d_ref[...] = a2d[:, None]
    ms_ref[...] = jnp.full((1, 8, 128), jnp.max(a2s), dtype=jnp.float32)
    md_ref[...] = jnp.full((1, 8, 128), jnp.max(a2d), dtype=jnp.float32)


def _p2_pallas(num, dent, bias1, lin2_w, att_src2, att_dst2):
    g = N // P_BN
    return pl.pallas_call(
        _p2_body,
        grid=(g,),
        in_specs=[
            pl.BlockSpec((8, P_BN, 64), lambda i: (0, i, 0)),
            pl.BlockSpec((P_BN, 8), lambda i: (i, 0)),
            pl.BlockSpec((1, 512), lambda i: (0, 0)),
            pl.BlockSpec((512, 100), lambda i: (0, 0)),
            pl.BlockSpec((1, 100), lambda i: (0, 0)),
            pl.BlockSpec((1, 100), lambda i: (0, 0)),
        ],
        out_specs=[
            pl.BlockSpec((P_BN, 112), lambda i: (i, 0)),
            pl.BlockSpec((P_BN, 1), lambda i: (i, 0)),
            pl.BlockSpec((P_BN, 1), lambda i: (i, 0)),
            pl.BlockSpec((1, 8, 128), lambda i: (i, 0, 0)),
            pl.BlockSpec((1, 8, 128), lambda i: (i, 0, 0)),
        ],
        out_shape=[
            jax.ShapeDtypeStruct((N, 112), jnp.float32),
            jax.ShapeDtypeStruct((N, 1), jnp.float32),
            jax.ShapeDtypeStruct((N, 1), jnp.float32),
            jax.ShapeDtypeStruct((N // P_BN, 8, 128), jnp.float32),
            jax.ShapeDtypeStruct((N // P_BN, 8, 128), jnp.float32),
        ],
    )(num, dent, bias1, lin2_w, att_src2, att_dst2)


def _e2_body(num_ref, den_ref, b2_ref, out_ref):
    x = num_ref[:, :100] / (den_ref[...] + 1e-16)
    out_ref[...] = jax.nn.relu(x + b2_ref[...])


def _e2_pallas(num, den, bias2):
    return pl.pallas_call(
        _e2_body,
        grid=(N // P_BN,),
        in_specs=[
            pl.BlockSpec((P_BN, 112), lambda i: (i, 0)),
            pl.BlockSpec((P_BN, 1), lambda i: (i, 0)),
            pl.BlockSpec((1, 100), lambda i: (0, 0)),
        ],
        out_specs=pl.BlockSpec((P_BN, 100), lambda i: (i, 0)),
        out_shape=jax.ShapeDtypeStruct((N, 100), jnp.float32),
    )(num, den, bias2)


def kernel(user_text, user_feats, graph_node_features, graph_edge_index,
           tweet_emb, fc_w1, fc_b1, fc_w2, fc_b2, gru_wi0, gru_wh0, gru_bi0,
           gru_bh0, gru_wi1, gru_wh1, gru_bi1, gru_bh1, lin1_w, att_src1,
           att_dst1, bias1, lin2_w, att_src2, att_dst2, bias2):
    user_embedding = jax.nn.relu(user_feats @ fc_w1 + fc_b1) @ fc_w2 + fc_b2

    tw = jnp.take(tweet_emb, graph_node_features, axis=0)
    tw = jnp.transpose(tw, (1, 0, 2))
    h0 = jax.random.normal(jax.random.key(42), (2, tw.shape[1], H),
                           dtype=jnp.float32)
    b0 = jnp.stack([gru_bi0, gru_bh0])
    b1 = jnp.stack([gru_bi1, gru_bh1])
    hn = _gru_pallas(tw, h0, gru_wi0.T, gru_wh0.T, b0, gru_wi1.T, gru_wh1.T,
                     b1)

    x_input = jnp.concatenate([hn[:BATCH], user_embedding, hn[BATCH:]], axis=0)

    idt = graph_edge_index.dtype
    loop = jnp.arange(N, dtype=idt)
    padz = jnp.zeros((E_PAD - E - N,), dtype=idt)
    src = jnp.concatenate([graph_edge_index[0], loop, padz]).astype(jnp.int32)
    dst = jnp.concatenate([graph_edge_index[1], loop, padz]).astype(jnp.int32)
    src = src.reshape(NW, NBLK, KB)
    dst = dst.reshape(NW, NBLK, KB)

    xh1, as1, ad1, ms1, md1 = _p1_pallas(
        x_input, lin1_w, att_src1.reshape(8, 64), att_dst1.reshape(8, 64))
    m1 = jnp.full((16,), jnp.max(ms1) + jnp.max(md1), dtype=jnp.float32)
    out_p1, den_p1 = _edge_l1(src, dst, xh1.reshape(8 * N, 64),
                              as1.T.reshape(8 * N), ad1.T.reshape(8 * N), m1)
    num1 = (out_p1[0] + out_p1[1])[:, :N]
    dent1 = (den_p1[0] + den_p1[1])[:, :N].T

    xh2, as2, ad2, ms2, md2 = _p2_pallas(
        num1, dent1, bias1.reshape(1, 512), lin2_w,
        att_src2.reshape(1, 100), att_dst2.reshape(1, 100))
    m2 = jnp.full((16,), jnp.max(ms2) + jnp.max(md2), dtype=jnp.float32)
    out_p2, den_p2 = _edge_l2(src, dst, xh2, as2.reshape(N), ad2.reshape(N),
                              m2)
    num2 = (out_p2[0, 0] + out_p2[1, 0])[:N]
    den2 = (den_p2[0, 0] + den_p2[1, 0])[:N]

    return _e2_pallas(num2, den2[:, None], bias2.reshape(1, 100))


# speedup vs baseline: 12.6665x; 1.1844x over previous
import functools
import jax
import jax.numpy as jnp
from jax import lax
from jax.experimental import pallas as pl
from jax.experimental.pallas import tpu as pltpu
from jax.experimental.pallas import tpu_sc as plsc

N = 10000
T = 20
D = 100
H = 100
BATCH = 1024
E = 160000

NC = 2
NS = 16
NW = NC * NS
KB = 128
E_PAD = 172032
EPW = E_PAD // NW
NBLK = EPW // KB
N_PAD = 10240
ROWS_PT = N_PAD // NS
ZR = 128

GRU_BN = 1000


def _gru_body(tw_ref, h0_ref, wi0_ref, wh0_ref, b0_ref, wi1_ref, wh1_ref,
              b1_ref, hn_ref):
    h1 = h0_ref[0]
    h2 = h0_ref[1]

    def step(t, carry):
        h1, h2 = carry

        def cell(x, h, wi_t, wh_t, bi, bh):
            gi = jnp.dot(x, wi_t, preferred_element_type=jnp.float32) + bi
            gh = jnp.dot(h, wh_t, preferred_element_type=jnp.float32) + bh
            r = jax.nn.sigmoid(gi[:, :H] + gh[:, :H])
            z = jax.nn.sigmoid(gi[:, H:2 * H] + gh[:, H:2 * H])
            n = jnp.tanh(gi[:, 2 * H:] + r * gh[:, 2 * H:])
            return (1.0 - z) * n + z * h

        x = tw_ref[t]
        h1 = cell(x, h1, wi0_ref[...], wh0_ref[...], b0_ref[0], b0_ref[1])
        h2 = cell(h1, h2, wi1_ref[...], wh1_ref[...], b1_ref[0], b1_ref[1])
        return (h1, h2)

    h1, h2 = jax.lax.fori_loop(0, T, step, (h1, h2))
    hn_ref[...] = h2


def _gru_pallas(tw, h0, wi0_t, wh0_t, b0, wi1_t, wh1_t, b1):
    nt = tw.shape[1]
    return pl.pallas_call(
        _gru_body,
        grid=(nt // GRU_BN,),
        in_specs=[
            pl.BlockSpec((T, GRU_BN, D), lambda i: (0, i, 0)),
            pl.BlockSpec((2, GRU_BN, H), lambda i: (0, i, 0)),
            pl.BlockSpec((D, 3 * H), lambda i: (0, 0)),
            pl.BlockSpec((H, 3 * H), lambda i: (0, 0)),
            pl.BlockSpec((2, 3 * H), lambda i: (0, 0)),
            pl.BlockSpec((H, 3 * H), lambda i: (0, 0)),
            pl.BlockSpec((H, 3 * H), lambda i: (0, 0)),
            pl.BlockSpec((2, 3 * H), lambda i: (0, 0)),
        ],
        out_specs=pl.BlockSpec((GRU_BN, H), lambda i: (i, 0)),
        out_shape=jax.ShapeDtypeStruct((nt, H), jnp.float32),
    )(tw, h0, wi0_t, wh0_t, b0, wi1_t, wh1_t, b1)


P_BN = 2000


def _p1_body(x_ref, w_ref, asr_ref, adr_ref, xh_ref, as_ref, ad_ref, ms_ref,
             md_ref):
    xh = jnp.dot(x_ref[...], w_ref[...], preferred_element_type=jnp.float32)
    a_s, a_d = [], []
    for h in range(8):
        xh_h = xh[:, h * 64:(h + 1) * 64]
        xh_ref[h] = xh_h
        a_s.append(jnp.sum(xh_h * asr_ref[h][None, :], axis=1)[:, None])
        a_d.append(jnp.sum(xh_h * adr_ref[h][None, :], axis=1)[:, None])
    a_s = jnp.concatenate(a_s, axis=1)
    a_d = jnp.concatenate(a_d, axis=1)
    as_ref[...] = a_s
    ad_ref[...] = a_d
    ms_ref[...] = jnp.full((1, 8, 128), jnp.max(a_s), dtype=jnp.float32)
    md_ref[...] = jnp.full((1, 8, 128), jnp.max(a_d), dtype=jnp.float32)


def _p1_pallas(x, lin_w, att_src, att_dst):
    n = x.shape[0]
    g = n // P_BN
    return pl.pallas_call(
        _p1_body,
        grid=(g,),
        in_specs=[
            pl.BlockSpec((P_BN, D), lambda i: (i, 0)),
            pl.BlockSpec((D, 512), lambda i: (0, 0)),
            pl.BlockSpec((8, 64), lambda i: (0, 0)),
            pl.BlockSpec((8, 64), lambda i: (0, 0)),
        ],
        out_specs=[
            pl.BlockSpec((8, P_BN, 64), lambda i: (0, i, 0)),
            pl.BlockSpec((P_BN, 8), lambda i: (i, 0)),
            pl.BlockSpec((P_BN, 8), lambda i: (i, 0)),
            pl.BlockSpec((1, 8, 128), lambda i: (i, 0, 0)),
            pl.BlockSpec((1, 8, 128), lambda i: (i, 0, 0)),
        ],
        out_shape=[
            jax.ShapeDtypeStruct((8, n, 64), jnp.float32),
            jax.ShapeDtypeStruct((n, 8), jnp.float32),
            jax.ShapeDtypeStruct((n, 8), jnp.float32),
            jax.ShapeDtypeStruct((g, 8, 128), jnp.float32),
            jax.ShapeDtypeStruct((g, 8, 128), jnp.float32),
        ],
    )(x, lin_w, att_src, att_dst)


def _make_edge_kernel(heads, c):
    mesh = plsc.VectorSubcoreMesh(core_axis_name="c", subcore_axis_name="s")

    @functools.partial(
        pl.kernel,
        mesh=mesh,
        compiler_params=pltpu.CompilerParams(use_tc_tiling_on_sc=False),
        out_type=[
            jax.ShapeDtypeStruct((NC, heads, N_PAD, c), jnp.float32),
            jax.ShapeDtypeStruct((NC, heads, N_PAD), jnp.float32),
        ],
        scratch_types=[
            pltpu.VMEM((NBLK, KB), jnp.int32),
            pltpu.VMEM((NBLK, KB), jnp.int32),
            [pltpu.VMEM((KB,), jnp.int32)] * 2,
            [pltpu.VMEM((KB,), jnp.int32)] * 2,
            [pltpu.VMEM((KB,), jnp.float32)] * 2,
            [pltpu.VMEM((KB,), jnp.float32)] * 2,
            [pltpu.VMEM((KB,), jnp.float32)] * 2,
            [pltpu.VMEM((KB, c), jnp.float32)] * 2,
            pltpu.VMEM((16,), jnp.float32),
            pltpu.VMEM((ZR, c), jnp.float32),
            pltpu.VMEM((ROWS_PT,), jnp.float32),
            pltpu.VMEM_SHARED((N_PAD, c), jnp.float32),
            pltpu.VMEM_SHARED((N_PAD,), jnp.float32),
            [pltpu.SemaphoreType.DMA] * 2,
            [pltpu.SemaphoreType.DMA] * 2,
            [pltpu.SemaphoreType.DMA] * 2,
            [pltpu.SemaphoreType.DMA] * 2,
            [pltpu.SemaphoreType.DMA] * 2,
        ],
    )
    def edge_kernel(src_hbm, dst_hbm, xh_hbm, as_hbm, ad_hbm, m_hbm,
                    out_hbm, den_hbm,
                    srcv, dstv, idxs, idxd, asg, adg, exq, rowb, mv,
                    zrow, zden, out_sh, den_sh, sem_s, sem_d, sem_r,
                    sem_ms, sem_dn):
        cid = lax.axis_index("c")
        sid = lax.axis_index("s")
        wid = sid * NC + cid
        ebase = wid * EPW

        pltpu.sync_copy(src_hbm.at[wid], srcv)
        pltpu.sync_copy(dst_hbm.at[wid], dstv)
        pltpu.sync_copy(m_hbm, mv)

        zv = jnp.zeros((16,), jnp.float32)

        def zrow_body(r, _):
            for j in range(c // 16):
                zrow[r, pl.ds(j * 16, 16)] = zv
            return 0
        lax.fori_loop(0, ZR, zrow_body, 0)

        def zden_body(i, _):
            zden[pl.ds(i * 16, 16)] = zv
            return 0
        lax.fori_loop(0, ROWS_PT // 16, zden_body, 0)

        mvec = mv[...]
        lanes = lax.iota(jnp.int32, 16)

        def fire_gathers(h, jb, b):
            def mk_idx(k, _):
                sv = srcv[jb, pl.ds(k * 16, 16)]
                dv = dstv[jb, pl.ds(k * 16, 16)]
                idxs[b][pl.ds(k * 16, 16)] = sv + h * N
                idxd[b][pl.ds(k * 16, 16)] = dv + h * N
                return 0
            lax.fori_loop(0, KB // 16, mk_idx, 0)
            pltpu.async_copy(as_hbm.at[idxs[b]], asg[b], sem_s[b])
            pltpu.async_copy(ad_hbm.at[idxd[b]], adg[b], sem_d[b])
            pltpu.async_copy(xh_hbm.at[idxs[b]], rowb[b], sem_r[b])

        def wait_gathers(b):
            pltpu.make_async_copy(as_hbm.at[idxs[b]], asg[b], sem_s[b]).wait()
            pltpu.make_async_copy(ad_hbm.at[idxd[b]], adg[b], sem_d[b]).wait()
            pltpu.make_async_copy(xh_hbm.at[idxs[b]], rowb[b], sem_r[b]).wait()

        def per_head(h, _):
            for z in range(ROWS_PT // ZR):
                pltpu.sync_copy(
                    zrow, out_sh.at[pl.ds(sid * ROWS_PT + z * ZR, ZR)])
            pltpu.sync_copy(zden, den_sh.at[pl.ds(sid * ROWS_PT, ROWS_PT)])
            plsc.subcore_barrier()

            fire_gathers(h, 0, 0)

            def outer(jo, _):
                for b in (0, 1):
                    jb = 2 * jo + b
                    nb = 1 - b

                    @pl.when(jb + 1 < NBLK)
                    def _():
                        @pl.when(jb >= 1)
                        def _():
                            pltpu.make_async_copy(
                                rowb[nb], out_sh.at[dstv.at[jb - 1]],
                                sem_ms[nb]).wait()
                        fire_gathers(h, jb + 1, nb)

                    wait_gathers(b)

                    @pl.when(jb >= 2)
                    def _():
                        pltpu.make_async_copy(
                            exq[b], den_sh.at[dstv.at[jb - 2]],
                            sem_dn[b]).wait()

                    def mk_ex(k, _):
                        al = (asg[b][pl.ds(k * 16, 16)]
                              + adg[b][pl.ds(k * 16, 16)])
                        al = jnp.where(al >= 0.0, al, al * 0.2)
                        ex = jnp.exp(al - mvec)
                        gid = ebase + jb * KB + k * 16 + lanes
                        ex = jnp.where(gid < E + N, ex, 0.0)
                        exq[b][pl.ds(k * 16, 16)] = ex
                        return 0
                    lax.fori_loop(0, KB // 16, mk_ex, 0)

                    pltpu.async_copy(exq[b], den_sh.at[dstv.at[jb]],
                                     sem_dn[b], add=True)

                    def scale(k, _):
                        exvec = exq[b][pl.ds(k * 16, 16)]
                        for e in range(16):
                            coef = jnp.full((16,), exvec[e],
                                            dtype=jnp.float32)
                            r = k * 16 + e
                            for j in range(c // 16):
                                rowb[b][r, pl.ds(j * 16, 16)] = (
                                    rowb[b][r, pl.ds(j * 16, 16)] * coef)
                        return 0
                    lax.fori_loop(0, KB // 16, scale, 0)

                    pltpu.async_copy(rowb[b], out_sh.at[dstv.at[jb]],
                                     sem_ms[b], add=True)
                return 0

            lax.fori_loop(0, NBLK // 2, outer, 0)

            pltpu.make_async_copy(
                rowb[0], out_sh.at[dstv.at[NBLK - 2]], sem_ms[0]).wait()
            pltpu.make_async_copy(
                rowb[1], out_sh.at[dstv.at[NBLK - 1]], sem_ms[1]).wait()
            pltpu.make_async_copy(
                exq[0], den_sh.at[dstv.at[NBLK - 2]], sem_dn[0]).wait()
            pltpu.make_async_copy(
                exq[1], den_sh.at[dstv.at[NBLK - 1]], sem_dn[1]).wait()
            plsc.subcore_barrier()

            pltpu.sync_copy(
                out_sh.at[pl.ds(sid * ROWS_PT, ROWS_PT)],
                out_hbm.at[cid, h, pl.ds(sid * ROWS_PT, ROWS_PT)])
            pltpu.sync_copy(
                den_sh.at[pl.ds(sid * ROWS_PT, ROWS_PT)],
                den_hbm.at[cid, h, pl.ds(sid * ROWS_PT, ROWS_PT)])
            return 0

        lax.fori_loop(0, heads, per_head, 0)

    return edge_kernel


_edge_l1 = _make_edge_kernel(8, 64)
_edge_l2 = _make_edge_kernel(1, 112)


def _p2_body(num_ref, den_ref, b1_ref, w2_ref, as2_ref, ad2_ref,
             xh2_ref, a2s_ref, a2d_ref, ms_ref, md_ref):
    x1 = []
    for h in range(8):
        dh = den_ref[:, h][:, None] + 1e-16
        x1.append(num_ref[h] / dh)
    x1 = jnp.concatenate(x1, axis=1)
    x1 = jax.nn.relu(x1 + b1_ref[...])
    xh2 = jnp.dot(x1, w2_ref[...], preferred_element_type=jnp.float32)
    a2s = jnp.sum(xh2 * as2_ref[...], axis=1)
    a2d = jnp.sum(xh2 * ad2_ref[...], axis=1)
    bn = xh2.shape[0]
    xh2_ref[...] = jnp.concatenate(
        [xh2, jnp.zeros((bn, 12), jnp.float32)], axis=1)
    a2s_ref[...] = a2s[:, None]
    a2d_ref[...] = a2d[:, None]
    ms_ref[...] = jnp.full((1, 8, 128), jnp.max(a2s), dtype=jnp.float32)
    md_ref[...] = jnp.full((1, 8, 128), jnp.max(a2d), dtype=jnp.float32)


def _p2_pallas(num, dent, bias1, lin2_w, att_src2, att_dst2):
    g = N // P_BN
    return pl.pallas_call(
        _p2_body,
        grid=(g,),
        in_specs=[
            pl.BlockSpec((8, P_BN, 64), lambda i: (0, i, 0)),
            pl.BlockSpec((P_BN, 8), lambda i: (i, 0)),
            pl.BlockSpec((1, 512), lambda i: (0, 0)),
            pl.BlockSpec((512, 100), lambda i: (0, 0)),
            pl.BlockSpec((1, 100), lambda i: (0, 0)),
            pl.BlockSpec((1, 100), lambda i: (0, 0)),
        ],
        out_specs=[
            pl.BlockSpec((P_BN, 112), lambda i: (i, 0)),
            pl.BlockSpec((P_BN, 1), lambda i: (i, 0)),
            pl.BlockSpec((P_BN, 1), lambda i: (i, 0)),
            pl.BlockSpec((1, 8, 128), lambda i: (i, 0, 0)),
            pl.BlockSpec((1, 8, 128), lambda i: (i, 0, 0)),
        ],
        out_shape=[
            jax.ShapeDtypeStruct((N, 112), jnp.float32),
            jax.ShapeDtypeStruct((N, 1), jnp.float32),
            jax.ShapeDtypeStruct((N, 1), jnp.float32),
            jax.ShapeDtypeStruct((N // P_BN, 8, 128), jnp.float32),
            jax.ShapeDtypeStruct((N // P_BN, 8, 128), jnp.float32),
        ],
    )(num, dent, bias1, lin2_w, att_src2, att_dst2)


def _e2_body(num_ref, den_ref, b2_ref, out_ref):
    x = num_ref[:, :100] / (den_ref[...] + 1e-16)
    out_ref[...] = jax.nn.relu(x + b2_ref[...])


def _e2_pallas(num, den, bias2):
    return pl.pallas_call(
        _e2_body,
        grid=(N // P_BN,),
        in_specs=[
            pl.BlockSpec((P_BN, 112), lambda i: (i, 0)),
            pl.BlockSpec((P_BN, 1), lambda i: (i, 0)),
            pl.BlockSpec((1, 100), lambda i: (0, 0)),
        ],
        out_specs=pl.BlockSpec((P_BN, 100), lambda i: (i, 0)),
        out_shape=jax.ShapeDtypeStruct((N, 100), jnp.float32),
    )(num, den, bias2)


def kernel(user_text, user_feats, graph_node_features, graph_edge_index,
           tweet_emb, fc_w1, fc_b1, fc_w2, fc_b2, gru_wi0, gru_wh0, gru_bi0,
           gru_bh0, gru_wi1, gru_wh1, gru_bi1, gru_bh1, lin1_w, att_src1,
           att_dst1, bias1, lin2_w, att_src2, att_dst2, bias2):
    user_embedding = jax.nn.relu(user_feats @ fc_w1 + fc_b1) @ fc_w2 + fc_b2

    tw = jnp.take(tweet_emb, graph_node_features, axis=0)
    tw = jnp.transpose(tw, (1, 0, 2))
    h0 = jax.random.normal(jax.random.key(42), (2, tw.shape[1], H),
                           dtype=jnp.float32)
    b0 = jnp.stack([gru_bi0, gru_bh0])
    b1 = jnp.stack([gru_bi1, gru_bh1])
    hn = _gru_pallas(tw, h0, gru_wi0.T, gru_wh0.T, b0, gru_wi1.T, gru_wh1.T,
                     b1)

    x_input = jnp.concatenate([hn[:BATCH], user_embedding, hn[BATCH:]], axis=0)

    idt = graph_edge_index.dtype
    loop = jnp.arange(N, dtype=idt)
    padz = jnp.zeros((E_PAD - E - N,), dtype=idt)
    src = jnp.concatenate([graph_edge_index[0], loop, padz]).astype(jnp.int32)
    dst = jnp.concatenate([graph_edge_index[1], loop, padz]).astype(jnp.int32)
    src = src.reshape(NW, NBLK, KB)
    dst = dst.reshape(NW, NBLK, KB)

    xh1, as1, ad1, ms1, md1 = _p1_pallas(
        x_input, lin1_w, att_src1.reshape(8, 64), att_dst1.reshape(8, 64))
    m1 = jnp.full((16,), jnp.max(ms1) + jnp.max(md1), dtype=jnp.float32)
    out_p1, den_p1 = _edge_l1(src, dst, xh1.reshape(8 * N, 64),
                              as1.T.reshape(8 * N), ad1.T.reshape(8 * N), m1)
    num1 = (out_p1[0] + out_p1[1])[:, :N]
    dent1 = (den_p1[0] + den_p1[1])[:, :N].T

    xh2, as2, ad2, ms2, md2 = _p2_pallas(
        num1, dent1, bias1.reshape(1, 512), lin2_w,
        att_src2.reshape(1, 100), att_dst2.reshape(1, 100))
    m2 = jnp.full((16,), jnp.max(ms2) + jnp.max(md2), dtype=jnp.float32)
    out_p2, den_p2 = _edge_l2(src, dst, xh2, as2.reshape(N), ad2.reshape(N),
                              m2)
    num2 = (out_p2[0, 0] + out_p2[1, 0])[:N]
    den2 = (den_p2[0, 0] + den_p2[1, 0])[:N]

    return _e2_pallas(num2, den2[:, None], bias2.reshape(1, 100))


# speedup vs baseline: 13.9591x; 1.1021x over previous
import functools
import jax
import jax.numpy as jnp
from jax import lax
from jax.experimental import pallas as pl
from jax.experimental.pallas import tpu as pltpu
from jax.experimental.pallas import tpu_sc as plsc

N = 10000
T = 20
D = 100
H = 100
BATCH = 1024
E = 160000

NC = 2
NS = 16
NW = NC * NS
KB = 128
E_PAD = 172032
EPW = E_PAD // NW
NBLK = EPW // KB
N_PAD = 10240
ROWS_PT = N_PAD // NS
ZR = 128

GRU_BN = 1000

NT = 8000
NR = NT * T
RPW = NR // NW
GFULL = RPW // KB
GTAIL = RPW - GFULL * KB


def _gru_body(tw_ref, h0_ref, wi0_ref, wh0_ref, b0_ref, wi1_ref, wh1_ref,
              b1_ref, hn_ref):
    h1 = h0_ref[0]
    h2 = h0_ref[1]

    def step(t, carry):
        h1, h2 = carry

        def cell(x, h, wi_t, wh_t, bi, bh):
            gi = jnp.dot(x, wi_t, preferred_element_type=jnp.float32) + bi
            gh = jnp.dot(h, wh_t, preferred_element_type=jnp.float32) + bh
            r = jax.nn.sigmoid(gi[:, :H] + gh[:, :H])
            z = jax.nn.sigmoid(gi[:, H:2 * H] + gh[:, H:2 * H])
            n = jnp.tanh(gi[:, 2 * H:] + r * gh[:, 2 * H:])
            return (1.0 - z) * n + z * h

        x = tw_ref[t][:, :D]
        h1 = cell(x, h1, wi0_ref[...], wh0_ref[...], b0_ref[0], b0_ref[1])
        h2 = cell(h1, h2, wi1_ref[...], wh1_ref[...], b1_ref[0], b1_ref[1])
        return (h1, h2)

    h1, h2 = jax.lax.fori_loop(0, T, step, (h1, h2))
    hn_ref[...] = h2


def _gru_pallas(tw, h0, wi0_t, wh0_t, b0, wi1_t, wh1_t, b1):
    nt = tw.shape[1]
    return pl.pallas_call(
        _gru_body,
        grid=(nt // GRU_BN,),
        in_specs=[
            pl.BlockSpec((T, GRU_BN, 112), lambda i: (0, i, 0)),
            pl.BlockSpec((2, GRU_BN, H), lambda i: (0, i, 0)),
            pl.BlockSpec((D, 3 * H), lambda i: (0, 0)),
            pl.BlockSpec((H, 3 * H), lambda i: (0, 0)),
            pl.BlockSpec((2, 3 * H), lambda i: (0, 0)),
            pl.BlockSpec((H, 3 * H), lambda i: (0, 0)),
            pl.BlockSpec((H, 3 * H), lambda i: (0, 0)),
            pl.BlockSpec((2, 3 * H), lambda i: (0, 0)),
        ],
        out_specs=pl.BlockSpec((GRU_BN, H), lambda i: (i, 0)),
        out_shape=jax.ShapeDtypeStruct((nt, H), jnp.float32),
    )(tw, h0, wi0_t, wh0_t, b0, wi1_t, wh1_t, b1)


P_BN = 2000


def _p1_body(x_ref, w_ref, asr_ref, adr_ref, xh_ref, as_ref, ad_ref, ms_ref,
             md_ref):
    xh = jnp.dot(x_ref[...], w_ref[...], preferred_element_type=jnp.float32)
    a_s, a_d = [], []
    for h in range(8):
        xh_h = xh[:, h * 64:(h + 1) * 64]
        xh_ref[h] = xh_h
        a_s.append(jnp.sum(xh_h * asr_ref[h][None, :], axis=1)[:, None])
        a_d.append(jnp.sum(xh_h * adr_ref[h][None, :], axis=1)[:, None])
    a_s = jnp.concatenate(a_s, axis=1)
    a_d = jnp.concatenate(a_d, axis=1)
    as_ref[...] = a_s
    ad_ref[...] = a_d
    ms_ref[...] = jnp.full((1, 8, 128), jnp.max(a_s), dtype=jnp.float32)
    md_ref[...] = jnp.full((1, 8, 128), jnp.max(a_d), dtype=jnp.float32)


def _p1_pallas(x, lin_w, att_src, att_dst):
    n = x.shape[0]
    g = n // P_BN
    return pl.pallas_call(
        _p1_body,
        grid=(g,),
        in_specs=[
            pl.BlockSpec((P_BN, D), lambda i: (i, 0)),
            pl.BlockSpec((D, 512), lambda i: (0, 0)),
            pl.BlockSpec((8, 64), lambda i: (0, 0)),
            pl.BlockSpec((8, 64), lambda i: (0, 0)),
        ],
        out_specs=[
            pl.BlockSpec((8, P_BN, 64), lambda i: (0, i, 0)),
            pl.BlockSpec((P_BN, 8), lambda i: (i, 0)),
            pl.BlockSpec((P_BN, 8), lambda i: (i, 0)),
            pl.BlockSpec((1, 8, 128), lambda i: (i, 0, 0)),
            pl.BlockSpec((1, 8, 128), lambda i: (i, 0, 0)),
        ],
        out_shape=[
            jax.ShapeDtypeStruct((8, n, 64), jnp.float32),
            jax.ShapeDtypeStruct((n, 8), jnp.float32),
            jax.ShapeDtypeStruct((n, 8), jnp.float32),
            jax.ShapeDtypeStruct((g, 8, 128), jnp.float32),
            jax.ShapeDtypeStruct((g, 8, 128), jnp.float32),
        ],
    )(x, lin_w, att_src, att_dst)


def _make_edge_kernel(heads, c):
    mesh = plsc.VectorSubcoreMesh(core_axis_name="c", subcore_axis_name="s")

    @functools.partial(
        pl.kernel,
        mesh=mesh,
        compiler_params=pltpu.CompilerParams(use_tc_tiling_on_sc=False),
        out_type=[
            jax.ShapeDtypeStruct((NC, heads, N_PAD, c), jnp.float32),
            jax.ShapeDtypeStruct((NC, heads, N_PAD), jnp.float32),
        ],
        scratch_types=[
            pltpu.VMEM((NBLK, KB), jnp.int32),
            pltpu.VMEM((NBLK, KB), jnp.int32),
            [pltpu.VMEM((KB,), jnp.int32)] * 2,
            [pltpu.VMEM((KB,), jnp.int32)] * 2,
            [pltpu.VMEM((KB,), jnp.float32)] * 2,
            [pltpu.VMEM((KB,), jnp.float32)] * 2,
            [pltpu.VMEM((KB,), jnp.float32)] * 2,
            [pltpu.VMEM((KB, c), jnp.float32)] * 2,
            pltpu.VMEM((16,), jnp.float32),
            pltpu.VMEM((ZR, c), jnp.float32),
            pltpu.VMEM((ROWS_PT,), jnp.float32),
            pltpu.VMEM_SHARED((N_PAD, c), jnp.float32),
            pltpu.VMEM_SHARED((N_PAD,), jnp.float32),
            [pltpu.SemaphoreType.DMA] * 2,
            [pltpu.SemaphoreType.DMA] * 2,
            [pltpu.SemaphoreType.DMA] * 2,
            [pltpu.SemaphoreType.DMA] * 2,
            [pltpu.SemaphoreType.DMA] * 2,
        ],
    )
    def edge_kernel(src_hbm, dst_hbm, xh_hbm, as_hbm, ad_hbm, m_hbm,
                    out_hbm, den_hbm,
                    srcv, dstv, idxs, idxd, asg, adg, exq, rowb, mv,
                    zrow, zden, out_sh, den_sh, sem_s, sem_d, sem_r,
                    sem_ms, sem_dn):
        cid = lax.axis_index("c")
        sid = lax.axis_index("s")
        wid = sid * NC + cid
        ebase = wid * EPW

        pltpu.sync_copy(src_hbm.at[wid], srcv)
        pltpu.sync_copy(dst_hbm.at[wid], dstv)
        pltpu.sync_copy(m_hbm, mv)

        zv = jnp.zeros((16,), jnp.float32)

        def zrow_body(r, _):
            for j in range(c // 16):
                zrow[r, pl.ds(j * 16, 16)] = zv
            return 0
        lax.fori_loop(0, ZR, zrow_body, 0)

        def zden_body(i, _):
            zden[pl.ds(i * 16, 16)] = zv
            return 0
        lax.fori_loop(0, ROWS_PT // 16, zden_body, 0)

        mvec = mv[...]
        lanes = lax.iota(jnp.int32, 16)

        def fire_gathers(h, jb, b):
            def mk_idx(k, _):
                sv = srcv[jb, pl.ds(k * 16, 16)]
                dv = dstv[jb, pl.ds(k * 16, 16)]
                idxs[b][pl.ds(k * 16, 16)] = sv + h * N
                idxd[b][pl.ds(k * 16, 16)] = dv + h * N
                return 0
            lax.fori_loop(0, KB // 16, mk_idx, 0)
            pltpu.async_copy(as_hbm.at[idxs[b]], asg[b], sem_s[b])
            pltpu.async_copy(ad_hbm.at[idxd[b]], adg[b], sem_d[b])
            pltpu.async_copy(xh_hbm.at[idxs[b]], rowb[b], sem_r[b])

        def wait_gathers(b):
            pltpu.make_async_copy(as_hbm.at[idxs[b]], asg[b], sem_s[b]).wait()
            pltpu.make_async_copy(ad_hbm.at[idxd[b]], adg[b], sem_d[b]).wait()
            pltpu.make_async_copy(xh_hbm.at[idxs[b]], rowb[b], sem_r[b]).wait()

        def per_head(h, _):
            for z in range(ROWS_PT // ZR):
                pltpu.sync_copy(
                    zrow, out_sh.at[pl.ds(sid * ROWS_PT + z * ZR, ZR)])
            pltpu.sync_copy(zden, den_sh.at[pl.ds(sid * ROWS_PT, ROWS_PT)])
            plsc.subcore_barrier()

            fire_gathers(h, 0, 0)

            def outer(jo, _):
                for b in (0, 1):
                    jb = 2 * jo + b
                    nb = 1 - b

                    @pl.when(jb + 1 < NBLK)
                    def _():
                        @pl.when(jb >= 1)
                        def _():
                            pltpu.make_async_copy(
                                rowb[nb], out_sh.at[dstv.at[jb - 1]],
                                sem_ms[nb]).wait()
                        fire_gathers(h, jb + 1, nb)

                    wait_gathers(b)

                    @pl.when(jb >= 2)
                    def _():
                        pltpu.make_async_copy(
                            exq[b], den_sh.at[dstv.at[jb - 2]],
                            sem_dn[b]).wait()

                    def mk_ex(k, _):
                        al = (asg[b][pl.ds(k * 16, 16)]
                              + adg[b][pl.ds(k * 16, 16)])
                        al = jnp.where(al >= 0.0, al, al * 0.2)
                        ex = jnp.exp(al - mvec)
                        gid = ebase + jb * KB + k * 16 + lanes
                        ex = jnp.where(gid < E + N, ex, 0.0)
                        exq[b][pl.ds(k * 16, 16)] = ex
                        return 0
                    lax.fori_loop(0, KB // 16, mk_ex, 0)

                    pltpu.async_copy(exq[b], den_sh.at[dstv.at[jb]],
                                     sem_dn[b], add=True)

                    def scale(k, _):
                        exvec = exq[b][pl.ds(k * 16, 16)]
                        for e in range(16):
                            coef = jnp.full((16,), exvec[e],
                                            dtype=jnp.float32)
                            r = k * 16 + e
                            for j in range(c // 16):
                                rowb[b][r, pl.ds(j * 16, 16)] = (
                                    rowb[b][r, pl.ds(j * 16, 16)] * coef)
                        return 0
                    lax.fori_loop(0, KB // 16, scale, 0)

                    pltpu.async_copy(rowb[b], out_sh.at[dstv.at[jb]],
                                     sem_ms[b], add=True)
                return 0

            lax.fori_loop(0, NBLK // 2, outer, 0)

            pltpu.make_async_copy(
                rowb[0], out_sh.at[dstv.at[NBLK - 2]], sem_ms[0]).wait()
            pltpu.make_async_copy(
                rowb[1], out_sh.at[dstv.at[NBLK - 1]], sem_ms[1]).wait()
            pltpu.make_async_copy(
                exq[0], den_sh.at[dstv.at[NBLK - 2]], sem_dn[0]).wait()
            pltpu.make_async_copy(
                exq[1], den_sh.at[dstv.at[NBLK - 1]], sem_dn[1]).wait()
            plsc.subcore_barrier()

            pltpu.sync_copy(
                out_sh.at[pl.ds(sid * ROWS_PT, ROWS_PT)],
                out_hbm.at[cid, h, pl.ds(sid * ROWS_PT, ROWS_PT)])
            pltpu.sync_copy(
                den_sh.at[pl.ds(sid * ROWS_PT, ROWS_PT)],
                den_hbm.at[cid, h, pl.ds(sid * ROWS_PT, ROWS_PT)])
            return 0

        lax.fori_loop(0, heads, per_head, 0)

    return edge_kernel


_edge_l1 = _make_edge_kernel(8, 64)
_edge_l2 = _make_edge_kernel(1, 112)


def _p2_body(num_ref, den_ref, b1_ref, w2_ref, as2_ref, ad2_ref,
             xh2_ref, a2s_ref, a2d_ref, ms_ref, md_ref):
    x1 = []
    for h in range(8):
        dh = den_ref[:, h][:, None] + 1e-16
        x1.append(num_ref[h] / dh)
    x1 = jnp.concatenate(x1, axis=1)
    x1 = jax.nn.relu(x1 + b1_ref[...])
    xh2 = jnp.dot(x1, w2_ref[...], preferred_element_type=jnp.float32)
    a2s = jnp.sum(xh2 * as2_ref[...], axis=1)
    a2d = jnp.sum(xh2 * ad2_ref[...], axis=1)
    bn = xh2.shape[0]
    xh2_ref[...] = jnp.concatenate(
        [xh2, jnp.zeros((bn, 12), jnp.float32)], axis=1)
    a2s_ref[...] = a2s[:, None]
    a2d_ref[...] = a2d[:, None]
    ms_ref[...] = jnp.full((1, 8, 128), jnp.max(a2s), dtype=jnp.float32)
    md_ref[...] = jnp.full((1, 8, 128), jnp.max(a2d), dtype=jnp.float32)


def _p2_pallas(num, dent, bias1, lin2_w, att_src2, att_dst2):
    g = N // P_BN
    return pl.pallas_call(
        _p2_body,
        grid=(g,),
        in_specs=[
            pl.BlockSpec((8, P_BN, 64), lambda i: (0, i, 0)),
            pl.BlockSpec((P_BN, 8), lambda i: (i, 0)),
            pl.BlockSpec((1, 512), lambda i: (0, 0)),
            pl.BlockSpec((512, 100), lambda i: (0, 0)),
            pl.BlockSpec((1, 100), lambda i: (0, 0)),
            pl.BlockSpec((1, 100), lambda i: (0, 0)),
        ],
        out_specs=[
            pl.BlockSpec((P_BN, 112), lambda i: (i, 0)),
            pl.BlockSpec((P_BN, 1), lambda i: (i, 0)),
            pl.BlockSpec((P_BN, 1), lambda i: (i, 0)),
            pl.BlockSpec((1, 8, 128), lambda i: (i, 0, 0)),
            pl.BlockSpec((1, 8, 128), lambda i: (i, 0, 0)),
        ],
        out_shape=[
            jax.ShapeDtypeStruct((N, 112), jnp.float32),
            jax.ShapeDtypeStruct((N, 1), jnp.float32),
            jax.ShapeDtypeStruct((N, 1), jnp.float32),
            jax.ShapeDtypeStruct((N // P_BN, 8, 128), jnp.float32),
            jax.ShapeDtypeStruct((N // P_BN, 8, 128), jnp.float32),
        ],
    )(num, dent, bias1, lin2_w, att_src2, att_dst2)


def _e2_body(num_ref, den_ref, b2_ref, out_ref):
    x = num_ref[:, :100] / (den_ref[...] + 1e-16)
    out_ref[...] = jax.nn.relu(x + b2_ref[...])


def _e2_pallas(num, den, bias2):
    return pl.pallas_call(
        _e2_body,
        grid=(N // P_BN,),
        in_specs=[
            pl.BlockSpec((P_BN, 112), lambda i: (i, 0)),
            pl.BlockSpec((P_BN, 1), lambda i: (i, 0)),
            pl.BlockSpec((1, 100), lambda i: (0, 0)),
        ],
        out_specs=pl.BlockSpec((P_BN, 100), lambda i: (i, 0)),
        out_shape=jax.ShapeDtypeStruct((N, 100), jnp.float32),
    )(num, den, bias2)


def _make_emb_gather():
    mesh = plsc.VectorSubcoreMesh(core_axis_name="c", subcore_axis_name="s")

    @functools.partial(
        pl.kernel,
        mesh=mesh,
        compiler_params=pltpu.CompilerParams(use_tc_tiling_on_sc=False),
        out_type=jax.ShapeDtypeStruct((NR, 112), jnp.float32),
        scratch_types=[
            pltpu.VMEM((GFULL, KB), jnp.int32),
            pltpu.VMEM((GTAIL,), jnp.int32),
            [pltpu.VMEM((KB, 112), jnp.float32)] * 2,
            pltpu.VMEM((GTAIL, 112), jnp.float32),
            [pltpu.SemaphoreType.DMA] * 2,
            [pltpu.SemaphoreType.DMA] * 2,
            pltpu.SemaphoreType.DMA,
        ],
    )
    def emb_gather(emb_hbm, idx_hbm, idxt_hbm, out_hbm,
                   idxv, idxt, rowb, rowt, sem_g, sem_w, sem_t):
        cid = lax.axis_index("c")
        sid = lax.axis_index("s")
        wid = sid * NC + cid
        base = wid * RPW

        pltpu.sync_copy(idx_hbm.at[wid], idxv)
        pltpu.sync_copy(idxt_hbm.at[wid], idxt)

        pltpu.async_copy(emb_hbm.at[idxv.at[0]], rowb[0], sem_g[0])

        def blk(jb, _):
            for b in (0, 1):
                j = 2 * jb + b
                nb = 1 - b

                @pl.when(j + 1 < GFULL)
                def _():
                    @pl.when(j >= 1)
                    def _():
                        pltpu.make_async_copy(
                            rowb[nb],
                            out_hbm.at[pl.ds(base + (j - 1) * KB, KB)],
                            sem_w[nb]).wait()
                    pltpu.async_copy(emb_hbm.at[idxv.at[j + 1]], rowb[nb],
                                     sem_g[nb])

                pltpu.make_async_copy(emb_hbm.at[idxv.at[j]], rowb[b],
                                      sem_g[b]).wait()
                pltpu.async_copy(rowb[b],
                                 out_hbm.at[pl.ds(base + j * KB, KB)],
                                 sem_w[b])
            return 0

        lax.fori_loop(0, GFULL // 2, blk, 0)
        j = GFULL - 1
        pltpu.make_async_copy(emb_hbm.at[idxv.at[j]], rowb[0],
                              sem_g[0]).wait()
        pltpu.async_copy(rowb[0], out_hbm.at[pl.ds(base + j * KB, KB)],
                         sem_w[0])

        pltpu.async_copy(emb_hbm.at[idxt], rowt, sem_t).wait()
        pltpu.sync_copy(rowt, out_hbm.at[pl.ds(base + GFULL * KB, GTAIL)])

        pltpu.make_async_copy(
            rowb[1], out_hbm.at[pl.ds(base + (j - 1) * KB, KB)],
            sem_w[1]).wait()
        pltpu.make_async_copy(
            rowb[0], out_hbm.at[pl.ds(base + j * KB, KB)], sem_w[0]).wait()

    return emb_gather


_emb_gather = _make_emb_gather()


def kernel(user_text, user_feats, graph_node_features, graph_edge_index,
           tweet_emb, fc_w1, fc_b1, fc_w2, fc_b2, gru_wi0, gru_wh0, gru_bi0,
           gru_bh0, gru_wi1, gru_wh1, gru_bi1, gru_bh1, lin1_w, att_src1,
           att_dst1, bias1, lin2_w, att_src2, att_dst2, bias2):
    user_embedding = jax.nn.relu(user_feats @ fc_w1 + fc_b1) @ fc_w2 + fc_b2

    tok_t = graph_node_features.T.reshape(NR).astype(jnp.int32)
    tok_t = tok_t.reshape(NW, RPW)
    tok_full = tok_t[:, :GFULL * KB].reshape(NW, GFULL, KB)
    tok_tail = tok_t[:, GFULL * KB:]

    emb_p = jnp.pad(tweet_emb, ((0, 0), (0, 12)))
    tw = _emb_gather(emb_p, tok_full, tok_tail).reshape(T, NT, 112)
    h0 = jax.random.normal(jax.random.key(42), (2, NT, H),
                           dtype=jnp.float32)
    b0 = jnp.stack([gru_bi0, gru_bh0])
    b1 = jnp.stack([gru_bi1, gru_bh1])
    hn = _gru_pallas(tw, h0, gru_wi0.T, gru_wh0.T, b0, gru_wi1.T, gru_wh1.T,
                     b1)

    x_input = jnp.concatenate([hn[:BATCH], user_embedding, hn[BATCH:]], axis=0)

    idt = graph_edge_index.dtype
    loop = jnp.arange(N, dtype=idt)
    padz = jnp.zeros((E_PAD - E - N,), dtype=idt)
    src = jnp.concatenate([graph_edge_index[0], loop, padz]).astype(jnp.int32)
    dst = jnp.concatenate([graph_edge_index[1], loop, padz]).astype(jnp.int32)
    src = src.reshape(NW, NBLK, KB)
    dst = dst.reshape(NW, NBLK, KB)

    xh1, as1, ad1, ms1, md1 = _p1_pallas(
        x_input, lin1_w, att_src1.reshape(8, 64), att_dst1.reshape(8, 64))
    m1 = jnp.full((16,), jnp.max(ms1) + jnp.max(md1), dtype=jnp.float32)
    out_p1, den_p1 = _edge_l1(src, dst, xh1.reshape(8 * N, 64),
                              as1.T.reshape(8 * N), ad1.T.reshape(8 * N), m1)
    num1 = (out_p1[0] + out_p1[1])[:, :N]
    dent1 = (den_p1[0] + den_p1[1])[:, :N].T

    xh2, as2, ad2, ms2, md2 = _p2_pallas(
        num1, dent1, bias1.reshape(1, 512), lin2_w,
        att_src2.reshape(1, 100), att_dst2.reshape(1, 100))
    m2 = jnp.full((16,), jnp.max(ms2) + jnp.max(md2), dtype=jnp.float32)
    out_p2, den_p2 = _edge_l2(src, dst, xh2, as2.reshape(N), ad2.reshape(N),
                              m2)
    num2 = (out_p2[0, 0] + out_p2[1, 0])[:N]
    den2 = (den_p2[0, 0] + den_p2[1, 0])[:N]

    return _e2_pallas(num2, den2[:, None], bias2.reshape(1, 100))
